# baseline (speedup 1.0000x reference)
import functools

import jax
import jax.numpy as jnp
from jax import lax
from jax.experimental import pallas as pl
from jax.experimental.pallas import tpu as pltpu
from jax.experimental.pallas import tpu_sc as plsc

K = 8
L = 16
NEG_INF = float("-inf")


def _mm_body(x_ref, w_ref, b_ref, o_ref):
    acc = jax.lax.dot_general(
        x_ref[...], w_ref[...],
        dimension_numbers=(((1,), (1,)), ((), ())),
        preferred_element_type=jnp.float32,
    )
    o_ref[...] = acc + b_ref[...]


def _scores_matmul(x2d, w, b, bm=1024, bn=512):
    m, d = x2d.shape
    p = w.shape[0]
    return pl.pallas_call(
        _mm_body,
        grid=(m // bm, p // bn),
        in_specs=[
            pl.BlockSpec((bm, d), lambda i, j: (i, 0)),
            pl.BlockSpec((bn, d), lambda i, j: (j, 0)),
            pl.BlockSpec((1, bn), lambda i, j: (0, j)),
        ],
        out_specs=pl.BlockSpec((bm, bn), lambda i, j: (i, j)),
        out_shape=jax.ShapeDtypeStruct((m, p), jnp.float32),
    )(x2d, w, b.reshape(1, p))


def _round_bf16(a):
    u = jax.lax.bitcast_convert_type(a, jnp.uint32)
    r = (u + jnp.uint32(0x7FFF) + ((u >> 16) & jnp.uint32(1))) & jnp.uint32(0xFFFF0000)
    return jax.lax.bitcast_convert_type(r, jnp.float32)


def _sc_router(scores, x2d, W_w, b_w):
    T, P = scores.shape
    D = x2d.shape[1]
    NW = 32
    TPW = T // NW
    NSEG = 64
    SEG = P // NSEG
    CH = SEG // L
    ND = D // L

    mesh = plsc.VectorSubcoreMesh(core_axis_name="c", subcore_axis_name="s")

    @functools.partial(
        pl.kernel,
        out_type=[
            jax.ShapeDtypeStruct((T * K,), jnp.int32),
            jax.ShapeDtypeStruct((T * K,), jnp.float32),
        ],
        mesh=mesh,
        compiler_params=pltpu.CompilerParams(needs_layout_passes=False),
        scratch_types=[
            pltpu.VMEM((2 * P,), jnp.float32),
            pltpu.VMEM((2 * D,), jnp.float32),
            pltpu.VMEM((2 * K, D), jnp.float32),
            pltpu.VMEM((P,), jnp.float32),
            pltpu.VMEM((2 * L,), jnp.int32),
            pltpu.VMEM((TPW * K,), jnp.int32),
            pltpu.VMEM((TPW * K,), jnp.float32),
            pltpu.SemaphoreType.DMA,
            pltpu.SemaphoreType.DMA,
            pltpu.SemaphoreType.DMA,
            pltpu.SemaphoreType.DMA,
            pltpu.SemaphoreType.DMA,
            pltpu.SemaphoreType.DMA,
        ],
    )
    def body(scores_hbm, x_hbm, ww_hbm, bw_hbm, idx_out, pw_out,
             srow, xrow, wrows, bwv, idxs, idx_st, pw_st,
             sem_s0, sem_s1, sem_x0, sem_x1, sem_g0, sem_g1):
        wid = lax.axis_index("s") * 2 + lax.axis_index("c")
        base = wid * TPW
        lanes = lax.broadcasted_iota(jnp.int32, (L,), 0)
        neg = jnp.full((L,), NEG_INF, jnp.float32)
        zero = jnp.zeros((L,), jnp.float32)
        izero = jnp.zeros((L,), jnp.int32)
        BIG = jnp.full((L,), 2 ** 30, jnp.int32)
        sem_s = (sem_s0, sem_s1)
        sem_x = (sem_x0, sem_x1)
        sem_g = (sem_g0, sem_g1)

        pltpu.sync_copy(bw_hbm, bwv)

        def srow_copy(t, bb):
            return pltpu.make_async_copy(
                scores_hbm.at[base + t], srow.at[pl.ds(bb * P, P)], sem_s[bb])

        def xrow_copy(t, bb):
            return pltpu.make_async_copy(
                x_hbm.at[base + t], xrow.at[pl.ds(bb * D, D)], sem_x[bb])

        def gather_copy(bb):
            return pltpu.make_async_copy(
                ww_hbm.at[idxs.at[pl.ds(bb * L, K)]],
                wrows.at[pl.ds(bb * K, K)], sem_g[bb])

        def topk(t, bb):
            soff = bb * P
            seg_base = [soff + lanes * SEG + g * (L * SEG) for g in range(4)]

            def segfold(j, ms):
                return tuple(
                    jnp.maximum(m, plsc.load_gather(srow, [sb + j]))
                    for m, sb in zip(ms, seg_base))

            m4 = lax.fori_loop(0, SEG, segfold, (neg, neg, neg, neg))

            def round_body(r, carry):
                m0, m1, m2, m3, ivec = carry
                M = jnp.max(jnp.maximum(jnp.maximum(m0, m1),
                                        jnp.maximum(m2, m3)))
                segid = jnp.min(jnp.minimum(
                    jnp.minimum(jnp.where(m0 == M, lanes, BIG),
                                jnp.where(m1 == M, lanes + L, BIG)),
                    jnp.minimum(jnp.where(m2 == M, lanes + 2 * L, BIG),
                                jnp.where(m3 == M, lanes + 3 * L, BIG))))
                ebase = segid * SEG
                off = soff + ebase

                def rescan(j, c):
                    acc, iacc = c
                    v = srow[pl.ds(off + j * L, L)]
                    gt = v > acc
                    return (jnp.where(gt, v, acc),
                            jnp.where(gt, ebase + j * L + lanes, iacc))

                acc, iacc = lax.fori_loop(0, CH, rescan, (neg, BIG))
                eidx = jnp.min(jnp.where(acc == M, iacc, BIG))
                ivec = jnp.where(lanes == r, eidx, ivec)
                plsc.store_scatter(srow, [jnp.full((L,), soff, jnp.int32) + eidx],
                                   neg, mask=lanes == 0)

                def refold(j, m):
                    return jnp.maximum(m, srow[pl.ds(off + j * L, L)])

                nm = jnp.max(lax.fori_loop(0, CH, refold, neg))
                g = segid // L
                lm = lanes == (segid % L)
                m0 = jnp.where(lm & (g == 0), nm, m0)
                m1 = jnp.where(lm & (g == 1), nm, m1)
                m2 = jnp.where(lm & (g == 2), nm, m2)
                m3 = jnp.where(lm & (g == 3), nm, m3)
                return m0, m1, m2, m3, ivec

            carry = m4 + (izero,)
            ivec = lax.fori_loop(0, K, round_body, carry)[4]
            idxs[pl.ds(bb * L, L)] = ivec
            plsc.store_scatter(idx_st, [t * K + lanes], ivec, mask=lanes < K)

        def dot(t, bb):
            xoff = bb * D

            def dotbody(c, accs):
                xv = xrow[pl.ds(xoff + c * L, L)]
                return tuple(
                    a + wrows[bb * K + r, pl.ds(c * L, L)] * xv
                    for r, a in enumerate(accs))

            accs = lax.fori_loop(0, ND, dotbody, (zero,) * K)
            ivec = idxs[pl.ds(bb * L, L)]
            pwv = plsc.load_gather(bwv, [ivec])
            for r in range(K):
                pwv = jnp.where(lanes == r, pwv + jnp.sum(accs[r]), pwv)
            plsc.store_scatter(pw_st, [t * K + lanes], pwv, mask=lanes < K)

        srow_copy(0, 0).start()

        def pair_body(i, _):
            for b in (0, 1):
                t = 2 * i + b
                srow_copy(t, b).wait()

                @pl.when(t + 1 < TPW)
                def _():
                    srow_copy(t + 1, b ^ 1).start()

                xrow_copy(t, b).start()
                topk(t, b)
                gather_copy(b).start()

                @pl.when(t >= 1)
                def _():
                    gather_copy(b ^ 1).wait()
                    xrow_copy(t - 1, b ^ 1).wait()
                    dot(t - 1, b ^ 1)
            return 0

        lax.fori_loop(0, TPW // 2, pair_body, 0)
        gather_copy(1).wait()
        xrow_copy(TPW - 1, 1).wait()
        dot(TPW - 1, 1)

        pltpu.sync_copy(idx_st, idx_out.at[pl.ds(base * K, TPW * K)])
        pltpu.sync_copy(pw_st, pw_out.at[pl.ds(base * K, TPW * K)])

    return body(scores, x2d, W_w, b_w)


def kernel(x, W_sel, b_sel, W_w, b_w):
    batch, seq, d = x.shape
    pool = W_sel.shape[0]
    x2d = x.reshape(batch * seq, d)
    scores = _scores_matmul(x2d, W_sel, b_sel)
    xr = _round_bf16(x2d)
    wr = _round_bf16(W_w)
    idx_flat, pw_flat = _sc_router(scores, xr, wr, b_w)
    return (
        idx_flat.reshape(batch, seq, K),
        pw_flat.reshape(batch, seq, K),
        scores.reshape(batch, seq, pool),
    )


# speedup vs baseline: 1.2378x; 1.2378x over previous
import functools

import jax
import jax.numpy as jnp
from jax import lax
from jax.experimental import pallas as pl
from jax.experimental.pallas import tpu as pltpu
from jax.experimental.pallas import tpu_sc as plsc

K = 8
L = 16
NEG_INF = float("-inf")


def _mm_body(x_ref, w_ref, b_ref, o_ref):
    acc = jax.lax.dot_general(
        x_ref[...], w_ref[...],
        dimension_numbers=(((1,), (1,)), ((), ())),
        preferred_element_type=jnp.float32,
    )
    o_ref[...] = acc + b_ref[...]


def _scores_matmul(x2d, w, b, bm=1024, bn=512):
    m, d = x2d.shape
    p = w.shape[0]
    return pl.pallas_call(
        _mm_body,
        grid=(m // bm, p // bn),
        in_specs=[
            pl.BlockSpec((bm, d), lambda i, j: (i, 0)),
            pl.BlockSpec((bn, d), lambda i, j: (j, 0)),
            pl.BlockSpec((1, bn), lambda i, j: (0, j)),
        ],
        out_specs=pl.BlockSpec((bm, bn), lambda i, j: (i, j)),
        out_shape=jax.ShapeDtypeStruct((m, p), jnp.float32),
    )(x2d, w, b.reshape(1, p))


def _round_bf16(a):
    u = jax.lax.bitcast_convert_type(a, jnp.uint32)
    r = (u + jnp.uint32(0x7FFF) + ((u >> 16) & jnp.uint32(1))) & jnp.uint32(0xFFFF0000)
    return jax.lax.bitcast_convert_type(r, jnp.float32)


def _sc_router(scores, x2d, W_w, b_w):
    T, P = scores.shape
    D = x2d.shape[1]
    NW = 32
    TPW = T // NW
    NSEG = 64
    SEG = P // NSEG
    CH = SEG // L
    ND = D // L

    mesh = plsc.VectorSubcoreMesh(core_axis_name="c", subcore_axis_name="s")

    @functools.partial(
        pl.kernel,
        out_type=[
            jax.ShapeDtypeStruct((T * K,), jnp.int32),
            jax.ShapeDtypeStruct((T * K,), jnp.float32),
        ],
        mesh=mesh,
        compiler_params=pltpu.CompilerParams(needs_layout_passes=False),
        scratch_types=[
            pltpu.VMEM((2 * P,), jnp.float32),
            pltpu.VMEM((2 * D,), jnp.float32),
            pltpu.VMEM((2 * K, D), jnp.float32),
            pltpu.VMEM((P,), jnp.float32),
            pltpu.VMEM((2 * L,), jnp.int32),
            pltpu.VMEM((TPW * K,), jnp.int32),
            pltpu.VMEM((TPW * K,), jnp.float32),
            pltpu.SemaphoreType.DMA,
            pltpu.SemaphoreType.DMA,
            pltpu.SemaphoreType.DMA,
            pltpu.SemaphoreType.DMA,
            pltpu.SemaphoreType.DMA,
            pltpu.SemaphoreType.DMA,
        ],
    )
    def body(scores_hbm, x_hbm, ww_hbm, bw_hbm, idx_out, pw_out,
             srow, xrow, wrows, bwv, idxs, idx_st, pw_st,
             sem_s0, sem_s1, sem_x0, sem_x1, sem_g0, sem_g1):
        wid = lax.axis_index("s") * 2 + lax.axis_index("c")
        base = wid * TPW
        lanes = lax.broadcasted_iota(jnp.int32, (L,), 0)
        neg = jnp.full((L,), NEG_INF, jnp.float32)
        zero = jnp.zeros((L,), jnp.float32)
        izero = jnp.zeros((L,), jnp.int32)
        BIG = jnp.full((L,), 2 ** 30, jnp.int32)
        sem_s = (sem_s0, sem_s1)
        sem_x = (sem_x0, sem_x1)
        sem_g = (sem_g0, sem_g1)

        pltpu.sync_copy(bw_hbm, bwv)

        def srow_copy(t, bb):
            return pltpu.make_async_copy(
                scores_hbm.at[base + t], srow.at[pl.ds(bb * P, P)], sem_s[bb])

        def xrow_copy(t, bb):
            return pltpu.make_async_copy(
                x_hbm.at[base + t], xrow.at[pl.ds(bb * D, D)], sem_x[bb])

        def gather_copy(bb):
            return pltpu.make_async_copy(
                ww_hbm.at[idxs.at[pl.ds(bb * L, K)]],
                wrows.at[pl.ds(bb * K, K)], sem_g[bb])

        def topk(t, bb):
            soff = bb * P
            seg_base = [soff + lanes * SEG + g * (L * SEG) for g in range(4)]

            def segfold(j, ms):
                return tuple(
                    jnp.maximum(m, plsc.load_gather(srow, [sb + j]))
                    for m, sb in zip(ms, seg_base))

            m4 = lax.fori_loop(0, SEG, segfold, (neg, neg, neg, neg))

            def round_body(r, carry):
                m0, m1, m2, m3, ivec = carry
                M = jnp.max(jnp.maximum(jnp.maximum(m0, m1),
                                        jnp.maximum(m2, m3)))
                segid = jnp.min(jnp.minimum(
                    jnp.minimum(jnp.where(m0 == M, lanes, BIG),
                                jnp.where(m1 == M, lanes + L, BIG)),
                    jnp.minimum(jnp.where(m2 == M, lanes + 2 * L, BIG),
                                jnp.where(m3 == M, lanes + 3 * L, BIG))))
                ebase = segid * SEG
                off = soff + ebase

                def rescan(j, c):
                    acc, iacc = c
                    v = srow[pl.ds(off + j * L, L)]
                    gt = v > acc
                    return (jnp.where(gt, v, acc),
                            jnp.where(gt, ebase + j * L + lanes, iacc))

                acc, iacc = lax.fori_loop(0, CH, rescan, (neg, BIG))
                eidx = jnp.min(jnp.where(acc == M, iacc, BIG))
                ivec = jnp.where(lanes == r, eidx, ivec)
                plsc.store_scatter(srow, [jnp.full((L,), soff, jnp.int32) + eidx],
                                   neg, mask=lanes == 0)

                def refold(j, m):
                    return jnp.maximum(m, srow[pl.ds(off + j * L, L)])

                nm = jnp.max(lax.fori_loop(0, CH, refold, neg))
                g = segid // L
                lm = lanes == (segid % L)
                m0 = jnp.where(lm & (g == 0), nm, m0)
                m1 = jnp.where(lm & (g == 1), nm, m1)
                m2 = jnp.where(lm & (g == 2), nm, m2)
                m3 = jnp.where(lm & (g == 3), nm, m3)
                return m0, m1, m2, m3, ivec

            carry = m4 + (izero,)
            ivec = lax.fori_loop(0, K, round_body, carry)[4]
            idxs[pl.ds(bb * L, L)] = ivec
            plsc.store_scatter(idx_st, [t * K + lanes], ivec, mask=lanes < K)

        def dot(t, bb):
            xoff = bb * D

            def dotbody(c, accs):
                xv = xrow[pl.ds(xoff + c * L, L)]
                return tuple(
                    a + wrows[bb * K + r, pl.ds(c * L, L)] * xv
                    for r, a in enumerate(accs))

            accs = lax.fori_loop(0, ND, dotbody, (zero,) * K)
            ivec = idxs[pl.ds(bb * L, L)]
            pwv = plsc.load_gather(bwv, [ivec])
            for r in range(K):
                pwv = jnp.where(lanes == r, pwv + jnp.sum(accs[r]), pwv)
            plsc.store_scatter(pw_st, [t * K + lanes], pwv, mask=lanes < K)

        srow_copy(0, 0).start()

        def pair_body(i, _):
            for b in (0, 1):
                t = 2 * i + b
                srow_copy(t, b).wait()

                @pl.when(t + 1 < TPW)
                def _():
                    srow_copy(t + 1, b ^ 1).start()

                xrow_copy(t, b).start()
                topk(t, b)
                gather_copy(b).start()

                @pl.when(t >= 1)
                def _():
                    gather_copy(b ^ 1).wait()
                    xrow_copy(t - 1, b ^ 1).wait()
                    dot(t - 1, b ^ 1)
            return 0

        lax.fori_loop(0, TPW // 2, pair_body, 0)
        gather_copy(1).wait()
        xrow_copy(TPW - 1, 1).wait()
        dot(TPW - 1, 1)

        pltpu.sync_copy(idx_st, idx_out.at[pl.ds(base * K, TPW * K)])
        pltpu.sync_copy(pw_st, pw_out.at[pl.ds(base * K, TPW * K)])

    return body(scores, x2d, W_w, b_w)


def kernel(x, W_sel, b_sel, W_w, b_w):
    batch, seq, d = x.shape
    pool = W_sel.shape[0]
    x2d = x.reshape(batch * seq, d)
    wr = _round_bf16(W_w)
    C = 4
    mc = (batch * seq) // C
    scores_p, idx_p, pw_p = [], [], []
    for c in range(C):
        xc = jax.lax.slice_in_dim(x2d, c * mc, (c + 1) * mc)
        sc = _scores_matmul(xc, W_sel, b_sel)
        ic, pc = _sc_router(sc, _round_bf16(xc), wr, b_w)
        scores_p.append(sc)
        idx_p.append(ic)
        pw_p.append(pc)
    scores = jnp.concatenate(scores_p, 0)
    idx_flat = jnp.concatenate(idx_p, 0)
    pw_flat = jnp.concatenate(pw_p, 0)
    return (
        idx_flat.reshape(batch, seq, K),
        pw_flat.reshape(batch, seq, K),
        scores.reshape(batch, seq, pool),
    )


# speedup vs baseline: 1.2398x; 1.0016x over previous
import functools

import jax
import jax.numpy as jnp
from jax import lax
from jax.experimental import pallas as pl
from jax.experimental.pallas import tpu as pltpu
from jax.experimental.pallas import tpu_sc as plsc

K = 8
L = 16
NEG_INF = float("-inf")


def _mm_body(x_ref, w_ref, b_ref, o_ref):
    acc = jax.lax.dot_general(
        x_ref[...], w_ref[...],
        dimension_numbers=(((1,), (1,)), ((), ())),
        preferred_element_type=jnp.float32,
    )
    o_ref[...] = acc + b_ref[...]


def _scores_matmul(x2d, w, b, bm=1024, bn=512):
    m, d = x2d.shape
    p = w.shape[0]
    return pl.pallas_call(
        _mm_body,
        grid=(m // bm, p // bn),
        in_specs=[
            pl.BlockSpec((bm, d), lambda i, j: (i, 0)),
            pl.BlockSpec((bn, d), lambda i, j: (j, 0)),
            pl.BlockSpec((1, bn), lambda i, j: (0, j)),
        ],
        out_specs=pl.BlockSpec((bm, bn), lambda i, j: (i, j)),
        out_shape=jax.ShapeDtypeStruct((m, p), jnp.float32),
    )(x2d, w, b.reshape(1, p))


def _round_bf16(a):
    u = jax.lax.bitcast_convert_type(a, jnp.uint32)
    r = (u + jnp.uint32(0x7FFF) + ((u >> 16) & jnp.uint32(1))) & jnp.uint32(0xFFFF0000)
    return jax.lax.bitcast_convert_type(r, jnp.float32)


def _sc_router(scores, x2d, W_w, b_w):
    T, P = scores.shape
    D = x2d.shape[1]
    NW = 32
    TPW = T // NW
    NSEG = 64
    SEG = P // NSEG
    CH = SEG // L
    ND = D // L

    mesh = plsc.VectorSubcoreMesh(core_axis_name="c", subcore_axis_name="s")

    @functools.partial(
        pl.kernel,
        out_type=[
            jax.ShapeDtypeStruct((T * K,), jnp.int32),
            jax.ShapeDtypeStruct((T * K,), jnp.float32),
        ],
        mesh=mesh,
        compiler_params=pltpu.CompilerParams(needs_layout_passes=False),
        scratch_types=[
            pltpu.VMEM((2 * P,), jnp.float32),
            pltpu.VMEM((2 * D,), jnp.float32),
            pltpu.VMEM((2 * K, D), jnp.float32),
            pltpu.VMEM((P,), jnp.float32),
            pltpu.VMEM((2 * L,), jnp.int32),
            pltpu.VMEM((TPW * K,), jnp.int32),
            pltpu.VMEM((TPW * K,), jnp.float32),
            pltpu.SemaphoreType.DMA,
            pltpu.SemaphoreType.DMA,
            pltpu.SemaphoreType.DMA,
            pltpu.SemaphoreType.DMA,
            pltpu.SemaphoreType.DMA,
            pltpu.SemaphoreType.DMA,
        ],
    )
    def body(scores_hbm, x_hbm, ww_hbm, bw_hbm, idx_out, pw_out,
             srow, xrow, wrows, bwv, idxs, idx_st, pw_st,
             sem_s0, sem_s1, sem_x0, sem_x1, sem_g0, sem_g1):
        wid = lax.axis_index("s") * 2 + lax.axis_index("c")
        base = wid * TPW
        lanes = lax.broadcasted_iota(jnp.int32, (L,), 0)
        neg = jnp.full((L,), NEG_INF, jnp.float32)
        zero = jnp.zeros((L,), jnp.float32)
        izero = jnp.zeros((L,), jnp.int32)
        BIG = jnp.full((L,), 2 ** 30, jnp.int32)
        sem_s = (sem_s0, sem_s1)
        sem_x = (sem_x0, sem_x1)
        sem_g = (sem_g0, sem_g1)

        pltpu.sync_copy(bw_hbm, bwv)

        def srow_copy(t, bb):
            return pltpu.make_async_copy(
                scores_hbm.at[base + t], srow.at[pl.ds(bb * P, P)], sem_s[bb])

        def xrow_copy(t, bb):
            return pltpu.make_async_copy(
                x_hbm.at[base + t], xrow.at[pl.ds(bb * D, D)], sem_x[bb])

        def gather_copy(bb):
            return pltpu.make_async_copy(
                ww_hbm.at[idxs.at[pl.ds(bb * L, K)]],
                wrows.at[pl.ds(bb * K, K)], sem_g[bb])

        def topk(t, bb):
            soff = bb * P
            seg_base = [soff + lanes * SEG + g * (L * SEG) for g in range(4)]

            def segfold(j, ms):
                return tuple(
                    jnp.maximum(m, plsc.load_gather(srow, [sb + j]))
                    for m, sb in zip(ms, seg_base))

            m4 = lax.fori_loop(0, SEG, segfold, (neg, neg, neg, neg))

            def round_body(r, carry):
                m0, m1, m2, m3, ivec = carry
                M = jnp.max(jnp.maximum(jnp.maximum(m0, m1),
                                        jnp.maximum(m2, m3)))
                segid = jnp.min(jnp.minimum(
                    jnp.minimum(jnp.where(m0 == M, lanes, BIG),
                                jnp.where(m1 == M, lanes + L, BIG)),
                    jnp.minimum(jnp.where(m2 == M, lanes + 2 * L, BIG),
                                jnp.where(m3 == M, lanes + 3 * L, BIG))))
                ebase = segid * SEG
                off = soff + ebase

                def rescan(j, c):
                    acc, iacc = c
                    v = srow[pl.ds(off + j * L, L)]
                    gt = v > acc
                    return (jnp.where(gt, v, acc),
                            jnp.where(gt, ebase + j * L + lanes, iacc))

                acc, iacc = lax.fori_loop(0, CH, rescan, (neg, BIG))
                eidx = jnp.min(jnp.where(acc == M, iacc, BIG))
                ivec = jnp.where(lanes == r, eidx, ivec)
                plsc.store_scatter(srow, [jnp.full((L,), soff, jnp.int32) + eidx],
                                   neg, mask=lanes == 0)

                def refold(j, m):
                    return jnp.maximum(m, srow[pl.ds(off + j * L, L)])

                nm = jnp.max(lax.fori_loop(0, CH, refold, neg))
                g = segid // L
                lm = lanes == (segid % L)
                m0 = jnp.where(lm & (g == 0), nm, m0)
                m1 = jnp.where(lm & (g == 1), nm, m1)
                m2 = jnp.where(lm & (g == 2), nm, m2)
                m3 = jnp.where(lm & (g == 3), nm, m3)
                return m0, m1, m2, m3, ivec

            carry = m4 + (izero,)
            ivec = lax.fori_loop(0, K, round_body, carry)[4]
            idxs[pl.ds(bb * L, L)] = ivec
            plsc.store_scatter(idx_st, [t * K + lanes], ivec, mask=lanes < K)

        def dot(t, bb):
            xoff = bb * D

            def dotbody(c, accs):
                xv = xrow[pl.ds(xoff + c * L, L)]
                return tuple(
                    a + wrows[bb * K + r, pl.ds(c * L, L)] * xv
                    for r, a in enumerate(accs))

            accs = lax.fori_loop(0, ND, dotbody, (zero,) * K)
            ivec = idxs[pl.ds(bb * L, L)]
            pwv = plsc.load_gather(bwv, [ivec])
            for r in range(K):
                pwv = jnp.where(lanes == r, pwv + jnp.sum(accs[r]), pwv)
            plsc.store_scatter(pw_st, [t * K + lanes], pwv, mask=lanes < K)

        srow_copy(0, 0).start()

        def pair_body(i, _):
            for b in (0, 1):
                t = 2 * i + b
                srow_copy(t, b).wait()

                @pl.when(t + 1 < TPW)
                def _():
                    srow_copy(t + 1, b ^ 1).start()

                xrow_copy(t, b).start()
                topk(t, b)
                gather_copy(b).start()

                @pl.when(t >= 1)
                def _():
                    gather_copy(b ^ 1).wait()
                    xrow_copy(t - 1, b ^ 1).wait()
                    dot(t - 1, b ^ 1)
            return 0

        lax.fori_loop(0, TPW // 2, pair_body, 0)
        gather_copy(1).wait()
        xrow_copy(TPW - 1, 1).wait()
        dot(TPW - 1, 1)

        pltpu.sync_copy(idx_st, idx_out.at[pl.ds(base * K, TPW * K)])
        pltpu.sync_copy(pw_st, pw_out.at[pl.ds(base * K, TPW * K)])

    return body(scores, x2d, W_w, b_w)


def kernel(x, W_sel, b_sel, W_w, b_w):
    batch, seq, d = x.shape
    pool = W_sel.shape[0]
    x2d = x.reshape(batch * seq, d)
    wr = _round_bf16(W_w)
    C = 8
    mc = (batch * seq) // C
    scores_p, idx_p, pw_p = [], [], []
    for c in range(C):
        xc = jax.lax.slice_in_dim(x2d, c * mc, (c + 1) * mc)
        sc = _scores_matmul(xc, W_sel, b_sel)
        ic, pc = _sc_router(sc, _round_bf16(xc), wr, b_w)
        scores_p.append(sc)
        idx_p.append(ic)
        pw_p.append(pc)
    scores = jnp.concatenate(scores_p, 0)
    idx_flat = jnp.concatenate(idx_p, 0)
    pw_flat = jnp.concatenate(pw_p, 0)
    return (
        idx_flat.reshape(batch, seq, K),
        pw_flat.reshape(batch, seq, K),
        scores.reshape(batch, seq, pool),
    )
